# baseline (speedup 1.0000x reference)
import functools

import jax
import jax.numpy as jnp
from jax import lax
from jax.experimental import pallas as pl
from jax.experimental.pallas import tpu as pltpu
from jax.experimental.pallas import tpu_sc as plsc

NC = 2
NS = 16
CH = 80
NBUF = 10


def _fill_rows(rows, n, w, val):
  def body(i, c):
    for j in range(w // 16):
      rows[i, pl.ds(j * 16, 16)] = jnp.full((16,), val, jnp.float32)
    return c
  lax.fori_loop(0, n, body, 0)


def _npad(n):
  return -(-n // (NS * CH)) * (NS * CH)


@functools.lru_cache(maxsize=None)
def _make_segsum(n, e, w):
  nw = NC * NS
  per_tile = e // nw
  assert per_tile * nw == e and per_tile % CH == 0 and w % 128 == 0
  n_chunks = per_tile // CH
  npad = _npad(n)
  rows_per_tile = npad // NS
  n_full = rows_per_tile // CH
  mesh = plsc.VectorSubcoreMesh(core_axis_name="c", subcore_axis_name="s")

  SB = 4
  LI = 1
  LG = 3
  n_outer = -(-(n_chunks + SB) // SB)

  @functools.partial(
      pl.kernel,
      out_type=jax.ShapeDtypeStruct((NC, npad, w), jnp.float32),
      mesh=mesh,
      scratch_types=(
          [pltpu.VMEM((CH,), jnp.int32) for _ in range(2 * SB)]
          + [pltpu.VMEM((CH, w), jnp.float32) for _ in range(SB)]
          + [pltpu.VMEM_SHARED((npad, w), jnp.float32)]
          + [pltpu.SemaphoreType.DMA for _ in range(3 * SB)]
      ),
  )
  def k(x_hbm, src_hbm, dst_hbm, out_hbm, *rest):
    idx_s = rest[:SB]
    idx_d = rest[SB:2 * SB]
    rows = rest[2 * SB:3 * SB]
    acc = rest[3 * SB]
    sem_i = rest[3 * SB + 1:3 * SB + 1 + SB]
    sem_g = rest[3 * SB + 1 + SB:3 * SB + 1 + 2 * SB]
    sem_s = rest[3 * SB + 1 + 2 * SB:3 * SB + 1 + 3 * SB]
    cid = lax.axis_index("c")
    sid = lax.axis_index("s")
    _fill_rows(rows[0], CH, w, 0.0)
    base = sid * rows_per_tile
    def zacc(i, c):
      pltpu.sync_copy(rows[0], acc.at[pl.ds(base + i * CH, CH)])
      return c
    lax.fori_loop(0, n_full, zacc, 0)
    plsc.subcore_barrier()
    ebase = (cid * NS + sid) * per_tile

    def outer(g, c):
      for b in range(SB):
        i = g * SB + b
        @pl.when(jnp.logical_and(i >= SB, i - SB < n_chunks))
        def _():
          pltpu.make_async_copy(rows[b], acc.at[idx_d[b]], sem_s[b]).wait()
        @pl.when(i < n_chunks)
        def _():
          e0 = ebase + i * CH
          pltpu.async_copy(src_hbm.at[pl.ds(e0, CH)], idx_s[b], sem_i[b])
          pltpu.async_copy(dst_hbm.at[pl.ds(e0, CH)], idx_d[b], sem_i[b])
        j = i - LI
        b2 = (b + SB - LI) % SB
        @pl.when(jnp.logical_and(j >= 0, j < n_chunks))
        def _():
          pltpu.make_async_copy(src_hbm.at[pl.ds(0, CH)], idx_s[b2],
                                sem_i[b2]).wait()
          pltpu.make_async_copy(dst_hbm.at[pl.ds(0, CH)], idx_d[b2],
                                sem_i[b2]).wait()
          pltpu.async_copy(x_hbm.at[idx_s[b2]], rows[b2], sem_g[b2])
        q = i - LG
        b3 = (b + SB - LG) % SB
        @pl.when(jnp.logical_and(q >= 0, q < n_chunks))
        def _():
          pltpu.make_async_copy(x_hbm.at[idx_s[b3]], rows[b3],
                                sem_g[b3]).wait()
          pltpu.async_copy(rows[b3], acc.at[idx_d[b3]], sem_s[b3], add=True)
      return c
    lax.fori_loop(0, n_outer, outer, 0)
    plsc.subcore_barrier()
    def wout(i, c):
      r0 = base + i * CH
      pltpu.sync_copy(acc.at[pl.ds(r0, CH)], out_hbm.at[cid, pl.ds(r0, CH)])
      return c
    lax.fori_loop(0, n_full, wout, 0)

  return k


@functools.lru_cache(maxsize=None)
def _make_counts(n, e, w):
  nw = NC * NS
  per_tile = e // nw
  assert per_tile * nw == e and per_tile % CH == 0 and w % 128 == 0
  n_chunks = per_tile // CH
  npad = _npad(n)
  rows_per_tile = npad // NS
  n_full = rows_per_tile // CH
  mesh = plsc.VectorSubcoreMesh(core_axis_name="c", subcore_axis_name="s")

  n_outer = -(-(n_chunks + NBUF) // NBUF)
  LAG = 5

  @functools.partial(
      pl.kernel,
      out_type=jax.ShapeDtypeStruct((NC, npad, w), jnp.float32),
      mesh=mesh,
      scratch_types=(
          [pltpu.VMEM((CH, w), jnp.float32)]
          + [pltpu.VMEM((CH,), jnp.int32) for _ in range(NBUF)]
          + [pltpu.VMEM_SHARED((npad, w), jnp.float32)]
          + [pltpu.SemaphoreType.DMA for _ in range(2 * NBUF)]
      ),
  )
  def k(dst_hbm, out_hbm, ones, *rest):
    idx_d = rest[:NBUF]
    acc = rest[NBUF]
    sem_i = rest[NBUF + 1:NBUF + 1 + NBUF]
    sem_s = rest[NBUF + 1 + NBUF:NBUF + 1 + 2 * NBUF]
    cid = lax.axis_index("c")
    sid = lax.axis_index("s")
    _fill_rows(ones, CH, w, 0.0)
    base = sid * rows_per_tile
    def zacc(i, c):
      pltpu.sync_copy(ones, acc.at[pl.ds(base + i * CH, CH)])
      return c
    lax.fori_loop(0, n_full, zacc, 0)
    plsc.subcore_barrier()
    _fill_rows(ones, CH, w, 1.0)
    ebase = (cid * NS + sid) * per_tile

    def outer(g, c):
      for b in range(NBUF):
        i = g * NBUF + b
        @pl.when(jnp.logical_and(i >= NBUF, i - NBUF < n_chunks))
        def _():
          pltpu.make_async_copy(ones, acc.at[idx_d[b]], sem_s[b]).wait()
        @pl.when(i < n_chunks)
        def _():
          pltpu.async_copy(dst_hbm.at[pl.ds(ebase + i * CH, CH)], idx_d[b],
                           sem_i[b])
        j = i - LAG
        b2 = (b + NBUF - LAG) % NBUF
        @pl.when(jnp.logical_and(j >= 0, j < n_chunks))
        def _():
          pltpu.make_async_copy(dst_hbm.at[pl.ds(0, CH)], idx_d[b2],
                                sem_i[b2]).wait()
          pltpu.async_copy(ones, acc.at[idx_d[b2]], sem_s[b2], add=True)
      return c
    lax.fori_loop(0, n_outer, outer, 0)
    plsc.subcore_barrier()
    def wout(i, c):
      r0 = base + i * CH
      pltpu.sync_copy(acc.at[pl.ds(r0, CH)], out_hbm.at[cid, pl.ds(r0, CH)])
      return c
    lax.fori_loop(0, n_full, wout, 0)

  return k


@functools.lru_cache(maxsize=None)
def _make_edge_combine(n, e, d):
  nw = NC * NS
  per_tile = e // nw
  assert per_tile * nw == e and per_tile % CH == 0
  n_chunks = per_tile // CH
  mesh = plsc.VectorSubcoreMesh(core_axis_name="c", subcore_axis_name="s")

  n_outer = -(-(n_chunks + NBUF) // NBUF)
  LAG1 = 4
  LAG2 = 8

  @functools.partial(
      pl.kernel,
      out_type=jax.ShapeDtypeStruct((e, d), jnp.float32),
      mesh=mesh,
      scratch_types=(
          [pltpu.VMEM((per_tile,), jnp.int32),
           pltpu.VMEM((per_tile,), jnp.int32)]
          + [pltpu.VMEM((CH, d), jnp.float32) for _ in range(NBUF)]
          + [pltpu.SemaphoreType.DMA for _ in range(2 * NBUF)]
      ),
  )
  def k(a_hbm, b_hbm, src_hbm, dst_hbm, out_hbm, idx_s, idx_d, *rest):
    rows = rest[:NBUF]
    sem_g = rest[NBUF:2 * NBUF]
    sem_w = rest[2 * NBUF:3 * NBUF]
    cid = lax.axis_index("c")
    sid = lax.axis_index("s")
    ebase = (cid * NS + sid) * per_tile
    pltpu.sync_copy(src_hbm.at[pl.ds(ebase, per_tile)], idx_s)
    pltpu.sync_copy(dst_hbm.at[pl.ds(ebase, per_tile)], idx_d)

    def outer(g, c):
      for b in range(NBUF):
        i = g * NBUF + b
        @pl.when(jnp.logical_and(i >= NBUF, i - NBUF < n_chunks))
        def _():
          pltpu.make_async_copy(rows[b], out_hbm.at[pl.ds(0, CH)],
                                sem_w[b]).wait()
        @pl.when(i < n_chunks)
        def _():
          pltpu.async_copy(a_hbm.at[idx_s.at[pl.ds(i * CH, CH)]], rows[b],
                           sem_g[b])
        j = i - LAG1
        b2 = (b + NBUF - LAG1) % NBUF
        @pl.when(jnp.logical_and(j >= 0, j < n_chunks))
        def _():
          pltpu.make_async_copy(a_hbm.at[idx_s.at[pl.ds(0, CH)]], rows[b2],
                                sem_g[b2]).wait()
          pltpu.async_copy(b_hbm.at[idx_d.at[pl.ds(j * CH, CH)]], rows[b2],
                           sem_g[b2], add=True)
        q = i - LAG2
        b3 = (b + NBUF - LAG2) % NBUF
        @pl.when(jnp.logical_and(q >= 0, q < n_chunks))
        def _():
          pltpu.make_async_copy(b_hbm.at[idx_d.at[pl.ds(0, CH)]], rows[b3],
                                sem_g[b3]).wait()
          pltpu.async_copy(rows[b3], out_hbm.at[pl.ds(ebase + q * CH, CH)],
                           sem_w[b3])
      return c
    lax.fori_loop(0, n_outer, outer, 0)

  return k


def _dot_t(x, w):
  return lax.dot_general(x, w, (((1,), (1,)), ((), ())),
                         preferred_element_type=jnp.float32)


@functools.lru_cache(maxsize=None)
def _make_layer1(n, din, dh):
  bn = 1000
  assert n % bn == 0

  def body(p_ref, c_ref, x_ref, wl_ref, bl_ref, wr_ref, h_ref, inv_ref):
    cnt = c_ref[0][:, 0:1] + c_ref[1][:, 0:1]
    inv = 1.0 / jnp.maximum(cnt, 1.0)
    mean = (p_ref[0] + p_ref[1]) * inv
    h = _dot_t(mean, wl_ref[...]) + bl_ref[...] + _dot_t(x_ref[...], wr_ref[...])
    h_ref[...] = jnp.maximum(h, 0.0)
    inv_ref[...] = inv

  grid = n // bn
  return pl.pallas_call(
      body,
      grid=(grid,),
      in_specs=[
          pl.BlockSpec((NC, bn, din), lambda i: (0, i, 0)),
          pl.BlockSpec((NC, bn, din), lambda i: (0, i, 0)),
          pl.BlockSpec((bn, din), lambda i: (i, 0)),
          pl.BlockSpec((dh, din), lambda i: (0, 0)),
          pl.BlockSpec((1, dh), lambda i: (0, 0)),
          pl.BlockSpec((dh, din), lambda i: (0, 0)),
      ],
      out_specs=[
          pl.BlockSpec((bn, dh), lambda i: (i, 0)),
          pl.BlockSpec((bn, 1), lambda i: (i, 0)),
      ],
      out_shape=[
          jax.ShapeDtypeStruct((n, dh), jnp.float32),
          jax.ShapeDtypeStruct((n, 1), jnp.float32),
      ],
  )


@functools.lru_cache(maxsize=None)
def _make_layer2(n, dh):
  bn = 1000
  assert n % bn == 0

  def body(p_ref, h_ref, inv_ref, wl_ref, bl_ref, wr_ref, wa_ref, wb_ref,
           a_ref, b_ref):
    mean = (p_ref[0] + p_ref[1]) * inv_ref[...]
    ne = _dot_t(mean, wl_ref[...]) + bl_ref[...] + _dot_t(h_ref[...], wr_ref[...])
    a_ref[...] = _dot_t(ne, wa_ref[...])
    b_ref[...] = _dot_t(ne, wb_ref[...])

  grid = n // bn
  full = lambda i: (0, 0)
  return pl.pallas_call(
      body,
      grid=(grid,),
      in_specs=[
          pl.BlockSpec((NC, bn, dh), lambda i: (0, i, 0)),
          pl.BlockSpec((bn, dh), lambda i: (i, 0)),
          pl.BlockSpec((bn, 1), lambda i: (i, 0)),
          pl.BlockSpec((dh, dh), full),
          pl.BlockSpec((1, dh), full),
          pl.BlockSpec((dh, dh), full),
          pl.BlockSpec((dh, dh), full),
          pl.BlockSpec((dh, dh), full),
      ],
      out_specs=[
          pl.BlockSpec((bn, dh), lambda i: (i, 0)),
          pl.BlockSpec((bn, dh), lambda i: (i, 0)),
      ],
      out_shape=[
          jax.ShapeDtypeStruct((n, dh), jnp.float32),
          jax.ShapeDtypeStruct((n, dh), jnp.float32),
      ],
  )


@functools.lru_cache(maxsize=None)
def _make_final(e, dh, ed):
  be = next(b for b in (6400, 3200, 1280, 640, 320, 160, 80) if e % b == 0)
  grid = e // be

  def body(g_ref, ea_ref, wc_ref, bm1_ref, wm2_ref, bm2_ref, out_ref):
    d = lax.dot_general(ea_ref[...], wc_ref[...], (((0,), (1,)), ((), ())),
                        preferred_element_type=jnp.float32)
    z = jnp.maximum(g_ref[...] + d + bm1_ref[...], 0.0)
    i = pl.program_id(0)
    out_ref[pl.ds(i * be, be)] = jnp.sum(z * wm2_ref[...], axis=1) + bm2_ref[0, 0]

  return pl.pallas_call(
      body,
      grid=(grid,),
      in_specs=[
          pl.BlockSpec((be, dh), lambda i: (i, 0)),
          pl.BlockSpec((ed, be), lambda i: (0, i)),
          pl.BlockSpec((dh, ed), lambda i: (0, 0)),
          pl.BlockSpec((1, dh), lambda i: (0, 0)),
          pl.BlockSpec((1, dh), lambda i: (0, 0)),
          pl.BlockSpec((1, 1), lambda i: (0, 0)),
      ],
      out_specs=pl.BlockSpec((e,), lambda i: (0,)),
      out_shape=jax.ShapeDtypeStruct((e,), jnp.float32),
  )


def kernel(x, edge_index, edge_attr, Wl1, bl1, Wr1, Wl2, bl2, Wr2, Wm1, bm1,
           Wm2, bm2):
  n, din = x.shape
  e = edge_index.shape[1]
  dh = Wl1.shape[0]
  ed = edge_attr.shape[1]
  src = edge_index[0]
  dst = edge_index[1]

  cnts = _make_counts(n, e, din)(dst)
  p1 = _make_segsum(n, e, din)(x, src, dst)
  h, inv = _make_layer1(n, din, dh)(p1, cnts, x, Wl1, bl1[None, :], Wr1)
  p2 = _make_segsum(n, e, dh)(h, src, dst)
  a, b = _make_layer2(n, dh)(p2, h, inv, Wl2, bl2[None, :], Wr2,
                             Wm1[:, :dh], Wm1[:, dh:2 * dh])
  eat = edge_attr.T
  nwch = NC * NS * CH
  e1 = (e // 2) // nwch * nwch
  halves = [(0, e1), (e1, e - e1)] if 0 < e1 < e else [(0, e)]
  outs = []
  for off, eh in halves:
    g = _make_edge_combine(n, eh, dh)(a, b, src[off:off + eh],
                                      dst[off:off + eh])
    outs.append(
        _make_final(eh, dh, ed)(g, eat[:, off:off + eh], Wm1[:, 2 * dh:],
                                bm1[None, :], Wm2[0][None, :], bm2[None, :]))
  return jnp.concatenate(outs) if len(outs) > 1 else outs[0]


# speedup vs baseline: 1.0605x; 1.0605x over previous
import functools

import jax
import jax.numpy as jnp
from jax import lax
from jax.experimental import pallas as pl
from jax.experimental.pallas import tpu as pltpu
from jax.experimental.pallas import tpu_sc as plsc

NC = 2
NS = 16
CH = 80
NBUF = 10


def _fill_rows(rows, n, w, val):
  def body(i, c):
    for j in range(w // 16):
      rows[i, pl.ds(j * 16, 16)] = jnp.full((16,), val, jnp.float32)
    return c
  lax.fori_loop(0, n, body, 0)


def _npad(n):
  return -(-n // (NS * CH)) * (NS * CH)


@functools.lru_cache(maxsize=None)
def _make_segsum(n, e, w):
  nw = NC * NS
  per_tile = e // nw
  assert per_tile * nw == e and per_tile % CH == 0 and w % 128 == 0
  n_chunks = per_tile // CH
  npad = _npad(n)
  rows_per_tile = npad // NS
  n_full = rows_per_tile // CH
  mesh = plsc.VectorSubcoreMesh(core_axis_name="c", subcore_axis_name="s")

  SB = 4
  LI = 1
  LG = 3
  n_outer = -(-(n_chunks + SB) // SB)

  @functools.partial(
      pl.kernel,
      out_type=jax.ShapeDtypeStruct((NC, npad, w), jnp.float32),
      mesh=mesh,
      scratch_types=(
          [pltpu.VMEM((CH,), jnp.int32) for _ in range(2 * SB)]
          + [pltpu.VMEM((CH, w), jnp.float32) for _ in range(SB)]
          + [pltpu.VMEM_SHARED((npad, w), jnp.float32)]
          + [pltpu.SemaphoreType.DMA for _ in range(3 * SB)]
      ),
  )
  def k(x_hbm, src_hbm, dst_hbm, out_hbm, *rest):
    idx_s = rest[:SB]
    idx_d = rest[SB:2 * SB]
    rows = rest[2 * SB:3 * SB]
    acc = rest[3 * SB]
    sem_i = rest[3 * SB + 1:3 * SB + 1 + SB]
    sem_g = rest[3 * SB + 1 + SB:3 * SB + 1 + 2 * SB]
    sem_s = rest[3 * SB + 1 + 2 * SB:3 * SB + 1 + 3 * SB]
    cid = lax.axis_index("c")
    sid = lax.axis_index("s")
    _fill_rows(rows[0], CH, w, 0.0)
    base = sid * rows_per_tile
    def zacc(i, c):
      pltpu.sync_copy(rows[0], acc.at[pl.ds(base + i * CH, CH)])
      return c
    lax.fori_loop(0, n_full, zacc, 0)
    plsc.subcore_barrier()
    ebase = (cid * NS + sid) * per_tile

    def outer(g, c):
      for b in range(SB):
        i = g * SB + b
        @pl.when(jnp.logical_and(i >= SB, i - SB < n_chunks))
        def _():
          pltpu.make_async_copy(rows[b], acc.at[idx_d[b]], sem_s[b]).wait()
        @pl.when(i < n_chunks)
        def _():
          e0 = ebase + i * CH
          pltpu.async_copy(src_hbm.at[pl.ds(e0, CH)], idx_s[b], sem_i[b])
          pltpu.async_copy(dst_hbm.at[pl.ds(e0, CH)], idx_d[b], sem_i[b])
        j = i - LI
        b2 = (b + SB - LI) % SB
        @pl.when(jnp.logical_and(j >= 0, j < n_chunks))
        def _():
          pltpu.make_async_copy(src_hbm.at[pl.ds(0, CH)], idx_s[b2],
                                sem_i[b2]).wait()
          pltpu.make_async_copy(dst_hbm.at[pl.ds(0, CH)], idx_d[b2],
                                sem_i[b2]).wait()
          pltpu.async_copy(x_hbm.at[idx_s[b2]], rows[b2], sem_g[b2])
        q = i - LG
        b3 = (b + SB - LG) % SB
        @pl.when(jnp.logical_and(q >= 0, q < n_chunks))
        def _():
          pltpu.make_async_copy(x_hbm.at[idx_s[b3]], rows[b3],
                                sem_g[b3]).wait()
          pltpu.async_copy(rows[b3], acc.at[idx_d[b3]], sem_s[b3], add=True)
      return c
    lax.fori_loop(0, n_outer, outer, 0)
    plsc.subcore_barrier()
    def wout(i, c):
      r0 = base + i * CH
      pltpu.sync_copy(acc.at[pl.ds(r0, CH)], out_hbm.at[cid, pl.ds(r0, CH)])
      return c
    lax.fori_loop(0, n_full, wout, 0)

  return k


@functools.lru_cache(maxsize=None)
def _make_counts(n, e, w):
  nw = NC * NS
  per_tile = e // nw
  assert per_tile * nw == e and per_tile % CH == 0 and w % 128 == 0
  n_chunks = per_tile // CH
  npad = _npad(n)
  rows_per_tile = npad // NS
  n_full = rows_per_tile // CH
  mesh = plsc.VectorSubcoreMesh(core_axis_name="c", subcore_axis_name="s")

  n_outer = -(-(n_chunks + NBUF) // NBUF)
  LAG = 5

  @functools.partial(
      pl.kernel,
      out_type=jax.ShapeDtypeStruct((NC, npad, w), jnp.float32),
      mesh=mesh,
      scratch_types=(
          [pltpu.VMEM((CH, w), jnp.float32)]
          + [pltpu.VMEM((CH,), jnp.int32) for _ in range(NBUF)]
          + [pltpu.VMEM_SHARED((npad, w), jnp.float32)]
          + [pltpu.SemaphoreType.DMA for _ in range(2 * NBUF)]
      ),
  )
  def k(dst_hbm, out_hbm, ones, *rest):
    idx_d = rest[:NBUF]
    acc = rest[NBUF]
    sem_i = rest[NBUF + 1:NBUF + 1 + NBUF]
    sem_s = rest[NBUF + 1 + NBUF:NBUF + 1 + 2 * NBUF]
    cid = lax.axis_index("c")
    sid = lax.axis_index("s")
    _fill_rows(ones, CH, w, 0.0)
    base = sid * rows_per_tile
    def zacc(i, c):
      pltpu.sync_copy(ones, acc.at[pl.ds(base + i * CH, CH)])
      return c
    lax.fori_loop(0, n_full, zacc, 0)
    plsc.subcore_barrier()
    _fill_rows(ones, CH, w, 1.0)
    ebase = (cid * NS + sid) * per_tile

    def outer(g, c):
      for b in range(NBUF):
        i = g * NBUF + b
        @pl.when(jnp.logical_and(i >= NBUF, i - NBUF < n_chunks))
        def _():
          pltpu.make_async_copy(ones, acc.at[idx_d[b]], sem_s[b]).wait()
        @pl.when(i < n_chunks)
        def _():
          pltpu.async_copy(dst_hbm.at[pl.ds(ebase + i * CH, CH)], idx_d[b],
                           sem_i[b])
        j = i - LAG
        b2 = (b + NBUF - LAG) % NBUF
        @pl.when(jnp.logical_and(j >= 0, j < n_chunks))
        def _():
          pltpu.make_async_copy(dst_hbm.at[pl.ds(0, CH)], idx_d[b2],
                                sem_i[b2]).wait()
          pltpu.async_copy(ones, acc.at[idx_d[b2]], sem_s[b2], add=True)
      return c
    lax.fori_loop(0, n_outer, outer, 0)
    plsc.subcore_barrier()
    def wout(i, c):
      r0 = base + i * CH
      pltpu.sync_copy(acc.at[pl.ds(r0, CH)], out_hbm.at[cid, pl.ds(r0, CH)])
      return c
    lax.fori_loop(0, n_full, wout, 0)

  return k


@functools.lru_cache(maxsize=None)
def _make_edge_combine(n, e, d):
  nw = NC * NS
  per_tile = e // nw
  assert per_tile * nw == e and per_tile % CH == 0
  n_chunks = per_tile // CH
  mesh = plsc.VectorSubcoreMesh(core_axis_name="c", subcore_axis_name="s")

  n_outer = -(-(n_chunks + NBUF) // NBUF)
  LAG1 = 4
  LAG2 = 8

  @functools.partial(
      pl.kernel,
      out_type=jax.ShapeDtypeStruct((e, d), jnp.float32),
      mesh=mesh,
      scratch_types=(
          [pltpu.VMEM((per_tile,), jnp.int32),
           pltpu.VMEM((per_tile,), jnp.int32)]
          + [pltpu.VMEM((CH, d), jnp.float32) for _ in range(NBUF)]
          + [pltpu.SemaphoreType.DMA for _ in range(2 * NBUF)]
      ),
  )
  def k(a_hbm, b_hbm, src_hbm, dst_hbm, out_hbm, idx_s, idx_d, *rest):
    rows = rest[:NBUF]
    sem_g = rest[NBUF:2 * NBUF]
    sem_w = rest[2 * NBUF:3 * NBUF]
    cid = lax.axis_index("c")
    sid = lax.axis_index("s")
    ebase = (cid * NS + sid) * per_tile
    pltpu.sync_copy(src_hbm.at[pl.ds(ebase, per_tile)], idx_s)
    pltpu.sync_copy(dst_hbm.at[pl.ds(ebase, per_tile)], idx_d)

    def outer(g, c):
      for b in range(NBUF):
        i = g * NBUF + b
        @pl.when(jnp.logical_and(i >= NBUF, i - NBUF < n_chunks))
        def _():
          pltpu.make_async_copy(rows[b], out_hbm.at[pl.ds(0, CH)],
                                sem_w[b]).wait()
        @pl.when(i < n_chunks)
        def _():
          pltpu.async_copy(a_hbm.at[idx_s.at[pl.ds(i * CH, CH)]], rows[b],
                           sem_g[b])
        j = i - LAG1
        b2 = (b + NBUF - LAG1) % NBUF
        @pl.when(jnp.logical_and(j >= 0, j < n_chunks))
        def _():
          pltpu.make_async_copy(a_hbm.at[idx_s.at[pl.ds(0, CH)]], rows[b2],
                                sem_g[b2]).wait()
          pltpu.async_copy(b_hbm.at[idx_d.at[pl.ds(j * CH, CH)]], rows[b2],
                           sem_g[b2], add=True)
        q = i - LAG2
        b3 = (b + NBUF - LAG2) % NBUF
        @pl.when(jnp.logical_and(q >= 0, q < n_chunks))
        def _():
          pltpu.make_async_copy(b_hbm.at[idx_d.at[pl.ds(0, CH)]], rows[b3],
                                sem_g[b3]).wait()
          pltpu.async_copy(rows[b3], out_hbm.at[pl.ds(ebase + q * CH, CH)],
                           sem_w[b3])
      return c
    lax.fori_loop(0, n_outer, outer, 0)

  return k


def _dot_t(x, w):
  return lax.dot_general(x, w, (((1,), (1,)), ((), ())),
                         preferred_element_type=jnp.float32)


@functools.lru_cache(maxsize=None)
def _make_layer1(n, din, dh):
  bn = 1000
  assert n % bn == 0

  def body(p_ref, c_ref, x_ref, wl_ref, bl_ref, wr_ref, h_ref, inv_ref):
    cnt = c_ref[0][:, 0:1] + c_ref[1][:, 0:1]
    inv = 1.0 / jnp.maximum(cnt, 1.0)
    mean = (p_ref[0] + p_ref[1]) * inv
    h = _dot_t(mean, wl_ref[...]) + bl_ref[...] + _dot_t(x_ref[...], wr_ref[...])
    h_ref[...] = jnp.maximum(h, 0.0)
    inv_ref[...] = inv

  grid = n // bn
  return pl.pallas_call(
      body,
      grid=(grid,),
      in_specs=[
          pl.BlockSpec((NC, bn, din), lambda i: (0, i, 0)),
          pl.BlockSpec((NC, bn, din), lambda i: (0, i, 0)),
          pl.BlockSpec((bn, din), lambda i: (i, 0)),
          pl.BlockSpec((dh, din), lambda i: (0, 0)),
          pl.BlockSpec((1, dh), lambda i: (0, 0)),
          pl.BlockSpec((dh, din), lambda i: (0, 0)),
      ],
      out_specs=[
          pl.BlockSpec((bn, dh), lambda i: (i, 0)),
          pl.BlockSpec((bn, 1), lambda i: (i, 0)),
      ],
      out_shape=[
          jax.ShapeDtypeStruct((n, dh), jnp.float32),
          jax.ShapeDtypeStruct((n, 1), jnp.float32),
      ],
  )


@functools.lru_cache(maxsize=None)
def _make_layer2(n, dh):
  bn = 1000
  assert n % bn == 0

  def body(p_ref, h_ref, inv_ref, wl_ref, bl_ref, wr_ref, wa_ref, wb_ref,
           a_ref, b_ref):
    mean = (p_ref[0] + p_ref[1]) * inv_ref[...]
    ne = _dot_t(mean, wl_ref[...]) + bl_ref[...] + _dot_t(h_ref[...], wr_ref[...])
    a_ref[...] = _dot_t(ne, wa_ref[...])
    b_ref[...] = _dot_t(ne, wb_ref[...])

  grid = n // bn
  full = lambda i: (0, 0)
  return pl.pallas_call(
      body,
      grid=(grid,),
      in_specs=[
          pl.BlockSpec((NC, bn, dh), lambda i: (0, i, 0)),
          pl.BlockSpec((bn, dh), lambda i: (i, 0)),
          pl.BlockSpec((bn, 1), lambda i: (i, 0)),
          pl.BlockSpec((dh, dh), full),
          pl.BlockSpec((1, dh), full),
          pl.BlockSpec((dh, dh), full),
          pl.BlockSpec((dh, dh), full),
          pl.BlockSpec((dh, dh), full),
      ],
      out_specs=[
          pl.BlockSpec((bn, dh), lambda i: (i, 0)),
          pl.BlockSpec((bn, dh), lambda i: (i, 0)),
      ],
      out_shape=[
          jax.ShapeDtypeStruct((n, dh), jnp.float32),
          jax.ShapeDtypeStruct((n, dh), jnp.float32),
      ],
  )


@functools.lru_cache(maxsize=None)
def _make_final(e, dh, ed):
  be = next(b for b in (6400, 3200, 1280, 640, 320, 160, 80) if e % b == 0)
  grid = e // be

  def body(g_ref, ea_ref, wc_ref, bm1_ref, wm2_ref, bm2_ref, out_ref):
    d = lax.dot_general(ea_ref[...], wc_ref[...], (((0,), (1,)), ((), ())),
                        preferred_element_type=jnp.float32)
    z = jnp.maximum(g_ref[...] + d + bm1_ref[...], 0.0)
    i = pl.program_id(0)
    out_ref[pl.ds(i * be, be)] = jnp.sum(z * wm2_ref[...], axis=1) + bm2_ref[0, 0]

  return pl.pallas_call(
      body,
      grid=(grid,),
      in_specs=[
          pl.BlockSpec((be, dh), lambda i: (i, 0)),
          pl.BlockSpec((ed, be), lambda i: (0, i)),
          pl.BlockSpec((dh, ed), lambda i: (0, 0)),
          pl.BlockSpec((1, dh), lambda i: (0, 0)),
          pl.BlockSpec((1, dh), lambda i: (0, 0)),
          pl.BlockSpec((1, 1), lambda i: (0, 0)),
      ],
      out_specs=pl.BlockSpec((e,), lambda i: (0,)),
      out_shape=jax.ShapeDtypeStruct((e,), jnp.float32),
  )


def kernel(x, edge_index, edge_attr, Wl1, bl1, Wr1, Wl2, bl2, Wr2, Wm1, bm1,
           Wm2, bm2):
  n, din = x.shape
  e = edge_index.shape[1]
  dh = Wl1.shape[0]
  ed = edge_attr.shape[1]
  src = edge_index[0]
  dst = edge_index[1]

  cnts = _make_counts(n, e, din)(dst)
  p1 = _make_segsum(n, e, din)(x, src, dst)
  h, inv = _make_layer1(n, din, dh)(p1, cnts, x, Wl1, bl1[None, :], Wr1)
  p2 = _make_segsum(n, e, dh)(h, src, dst)
  a, b = _make_layer2(n, dh)(p2, h, inv, Wl2, bl2[None, :], Wr2,
                             Wm1[:, :dh], Wm1[:, dh:2 * dh])
  g = _make_edge_combine(n, e, dh)(a, b, src, dst)
  out = _make_final(e, dh, ed)(g, edge_attr.T, Wm1[:, 2 * dh:], bm1[None, :],
                               Wm2[0][None, :], bm2[None, :])
  return out
